# baseline (speedup 1.0000x reference)
import functools

import jax
import jax.numpy as jnp
from jax.experimental import pallas as pl
from jax.experimental.pallas import tpu as pltpu

E = 8
D = 128
FUSION = 128


def _moe_body(feat_ref, gw_ref, gb_ref, ew_ref, eb_ref, rep_ref, out_ref, *,
              bn):
    h = bn // 2
    for j in range(2):
        _moe_half(feat_ref, gw_ref, gb_ref, ew_ref, eb_ref, rep_ref, out_ref,
                  lo=j * h, bn=h)


def _moe_half(feat_ref, gw_ref, gb_ref, ew_ref, eb_ref, rep_ref, out_ref, *,
              lo, bn):
    feats = [feat_ref[e, lo:lo + bn] for e in range(E)]

    logits = jnp.dot(feats[0], gw_ref[0], preferred_element_type=jnp.float32)
    for e in range(1, E):
        logits = logits + jnp.dot(feats[e], gw_ref[e],
                                  preferred_element_type=jnp.float32)
    lt = jnp.transpose(logits) + gb_ref[:]
    m = jnp.max(lt, axis=0, keepdims=True)
    p = jnp.exp(lt - m)
    gates = p * (1.0 / jnp.sum(p, axis=0, keepdims=True))

    eidx = jax.lax.broadcasted_iota(jnp.int32, (E, bn), 0)
    m1 = jnp.max(gates, axis=0, keepdims=True)
    i1 = jnp.min(jnp.where(gates == m1, eidx, E), axis=0, keepdims=True)
    g2 = jnp.where(eidx == i1, -jnp.inf, gates)
    m2 = jnp.max(g2, axis=0, keepdims=True)
    i2 = jnp.min(jnp.where(g2 == m2, eidx, E), axis=0, keepdims=True)
    wt = jnp.where((eidx == i1) | (eidx == i2), gates, 0.0)
    w = jnp.transpose(wt)

    wrep = jnp.dot(w, rep_ref[:], preferred_element_type=jnp.float32)
    out = jnp.dot(w, eb_ref[:], preferred_element_type=jnp.float32)
    for e in range(E):
        out = out + jnp.dot(wrep[:, e * D:(e + 1) * D] * feats[e], ew_ref[e],
                            preferred_element_type=jnp.float32)
    out_ref[lo:lo + bn, :] = out


@jax.jit
def kernel(feat_list, gate_W, gate_b, expert_W, expert_b):
    E_, N_, D_ = feat_list.shape
    F = expert_W.shape[2]
    bn = 1024
    grid = (N_ // bn,)

    rep = jnp.repeat(jnp.eye(E_, dtype=jnp.float32), D_, axis=1)

    body = functools.partial(_moe_body, bn=bn)
    return pl.pallas_call(
        body,
        grid=grid,
        in_specs=[
            pl.BlockSpec((E_, bn, D_), lambda i: (0, i, 0)),
            pl.BlockSpec((E_, D_, E_), lambda i: (0, 0, 0)),
            pl.BlockSpec((E_, 1), lambda i: (0, 0)),
            pl.BlockSpec((E_, D_, F), lambda i: (0, 0, 0)),
            pl.BlockSpec((E_, F), lambda i: (0, 0)),
            pl.BlockSpec((E_, E_ * D_), lambda i: (0, 0)),
        ],
        out_specs=pl.BlockSpec((bn, F), lambda i: (i, 0)),
        out_shape=jax.ShapeDtypeStruct((N_, F), jnp.float32),
        compiler_params=pltpu.CompilerParams(
            dimension_semantics=("arbitrary",),
        ),
    )(feat_list, gate_W.reshape(E_, D_, E_), gate_b.reshape(E_, 1),
      expert_W, expert_b, rep)


# speedup vs baseline: 1.1319x; 1.1319x over previous
import functools

import jax
import jax.numpy as jnp
from jax.experimental import pallas as pl
from jax.experimental.pallas import tpu as pltpu

E = 8
D = 128
FUSION = 128


def _gating(feat_ref, gw_ref, lo, bn):
    feats = [feat_ref[e, lo:lo + bn] for e in range(E)]
    xu = jnp.concatenate(feats, axis=1)
    logits = jnp.dot(xu, gw_ref[:], preferred_element_type=jnp.float32)
    return feats, logits


def _topk_mask(logits, gb_ref, bn):
    lt = jnp.transpose(logits) + gb_ref[:]
    m = jnp.max(lt, axis=0, keepdims=True)
    p = jnp.exp(lt - m)
    gates = p * (1.0 / jnp.sum(p, axis=0, keepdims=True))

    eidx = jax.lax.broadcasted_iota(jnp.int32, (E, bn), 0)
    m1 = jnp.max(gates, axis=0, keepdims=True)
    i1 = jnp.min(jnp.where(gates == m1, eidx, E), axis=0, keepdims=True)
    g2 = jnp.where(eidx == i1, -jnp.inf, gates)
    m2 = jnp.max(g2, axis=0, keepdims=True)
    i2 = jnp.min(jnp.where(g2 == m2, eidx, E), axis=0, keepdims=True)
    wt = jnp.where((eidx == i1) | (eidx == i2), gates, 0.0)
    return jnp.transpose(wt)


def _mix(feats, w, ew_ref, eb_ref):
    out = jnp.dot(w, eb_ref[:], preferred_element_type=jnp.float32)
    for e in range(E):
        out = out + jnp.dot(w[:, e:e + 1] * feats[e], ew_ref[e],
                            preferred_element_type=jnp.float32)
    return out


def _moe_body(feat_ref, gw_ref, gb_ref, ew_ref, eb_ref, out_ref, *, bn):
    h = bn // 2
    fl = [None, None]
    lg = [None, None]
    for j in range(2):
        fl[j], lg[j] = _gating(feat_ref, gw_ref, j * h, h)
    ws = [_topk_mask(lg[j], gb_ref, h) for j in range(2)]
    for j in range(2):
        out_ref[j * h:(j + 1) * h, :] = _mix(fl[j], ws[j], ew_ref, eb_ref)


@jax.jit
def kernel(feat_list, gate_W, gate_b, expert_W, expert_b):
    E_, N_, D_ = feat_list.shape
    F = expert_W.shape[2]
    bn = 1024
    grid = (N_ // bn,)

    body = functools.partial(_moe_body, bn=bn)
    return pl.pallas_call(
        body,
        grid=grid,
        in_specs=[
            pl.BlockSpec((E_, bn, D_), lambda i: (0, i, 0)),
            pl.BlockSpec((E_ * D_, E_), lambda i: (0, 0)),
            pl.BlockSpec((E_, 1), lambda i: (0, 0)),
            pl.BlockSpec((E_, D_, F), lambda i: (0, 0, 0)),
            pl.BlockSpec((E_, F), lambda i: (0, 0)),
        ],
        out_specs=pl.BlockSpec((bn, F), lambda i: (i, 0)),
        out_shape=jax.ShapeDtypeStruct((N_, F), jnp.float32),
        compiler_params=pltpu.CompilerParams(
            dimension_semantics=("arbitrary",),
        ),
    )(feat_list, gate_W, gate_b.reshape(E_, 1), expert_W, expert_b)


# speedup vs baseline: 1.1610x; 1.0257x over previous
import functools

import jax
import jax.numpy as jnp
from jax.experimental import pallas as pl
from jax.experimental.pallas import tpu as pltpu

E = 8
D = 128
FUSION = 128


def _gating(feat_ref, gw_ref, lo, bn):
    feats = [feat_ref[e, lo:lo + bn] for e in range(E)]
    xu = jnp.concatenate(feats, axis=1)
    logits = jnp.dot(xu, gw_ref[:], preferred_element_type=jnp.float32)
    return feats, logits


def _topk_mask(logits, gb_ref, bn):
    lt = jnp.transpose(logits) + gb_ref[:]
    m = jnp.max(lt, axis=0, keepdims=True)
    p = jnp.exp(lt - m)
    gates = p * (1.0 / jnp.sum(p, axis=0, keepdims=True))

    eidx = jax.lax.broadcasted_iota(jnp.int32, (E, bn), 0)
    m1 = jnp.max(gates, axis=0, keepdims=True)
    i1 = jnp.min(jnp.where(gates == m1, eidx, E), axis=0, keepdims=True)
    g2 = jnp.where(eidx == i1, -jnp.inf, gates)
    m2 = jnp.max(g2, axis=0, keepdims=True)
    i2 = jnp.min(jnp.where(g2 == m2, eidx, E), axis=0, keepdims=True)
    wt = jnp.where((eidx == i1) | (eidx == i2), gates, 0.0)
    return jnp.transpose(wt)


def _mix(feats, w, ew_ref, eb_ref):
    out = jnp.dot(w, eb_ref[:], preferred_element_type=jnp.float32)
    for e in range(E):
        out = out + jnp.dot(w[:, e:e + 1] * feats[e], ew_ref[e],
                            preferred_element_type=jnp.float32)
    return out


def _moe_body(feat_ref, gw_ref, gb_ref, ew_ref, eb_ref, out_ref, *, bn):
    h = bn // 2
    fl = [None, None]
    lg = [None, None]
    for j in range(2):
        fl[j], lg[j] = _gating(feat_ref, gw_ref, j * h, h)
    w0 = _topk_mask(lg[0], gb_ref, h)
    out_ref[0:h, :] = _mix(fl[0], w0, ew_ref, eb_ref)
    w1 = _topk_mask(lg[1], gb_ref, h)
    out_ref[h:2 * h, :] = _mix(fl[1], w1, ew_ref, eb_ref)


@jax.jit
def kernel(feat_list, gate_W, gate_b, expert_W, expert_b):
    E_, N_, D_ = feat_list.shape
    F = expert_W.shape[2]
    bn = 1024
    grid = (N_ // bn,)

    body = functools.partial(_moe_body, bn=bn)
    return pl.pallas_call(
        body,
        grid=grid,
        in_specs=[
            pl.BlockSpec((E_, bn, D_), lambda i: (0, i, 0)),
            pl.BlockSpec((E_ * D_, E_), lambda i: (0, 0)),
            pl.BlockSpec((E_, 1), lambda i: (0, 0)),
            pl.BlockSpec((E_, D_, F), lambda i: (0, 0, 0)),
            pl.BlockSpec((E_, F), lambda i: (0, 0)),
        ],
        out_specs=pl.BlockSpec((bn, F), lambda i: (i, 0)),
        out_shape=jax.ShapeDtypeStruct((N_, F), jnp.float32),
        compiler_params=pltpu.CompilerParams(
            dimension_semantics=("arbitrary",),
        ),
    )(feat_list, gate_W, gate_b.reshape(E_, 1), expert_W, expert_b)


# speedup vs baseline: 1.2873x; 1.1088x over previous
import functools

import jax
import jax.numpy as jnp
from jax.experimental import pallas as pl
from jax.experimental.pallas import tpu as pltpu

E = 8
D = 128
FUSION = 128


def _gating(feat_ref, gw_ref, lo, bn):
    feats = [feat_ref[e, lo:lo + bn] for e in range(E)]
    xu = jnp.concatenate(feats, axis=1)
    logits = jnp.dot(xu, gw_ref[:], preferred_element_type=jnp.float32)
    return feats, logits


def _topk_mask(logits, gb_ref, bn):
    lt = jnp.transpose(logits) + gb_ref[:]
    m = jnp.max(lt, axis=0, keepdims=True)
    p = jnp.exp(lt - m)
    gates = p * (1.0 / jnp.sum(p, axis=0, keepdims=True))

    eidx = jax.lax.broadcasted_iota(jnp.int32, (E, bn), 0)
    m1 = jnp.max(gates, axis=0, keepdims=True)
    i1 = jnp.min(jnp.where(gates == m1, eidx, E), axis=0, keepdims=True)
    g2 = jnp.where(eidx == i1, -jnp.inf, gates)
    m2 = jnp.max(g2, axis=0, keepdims=True)
    i2 = jnp.min(jnp.where(g2 == m2, eidx, E), axis=0, keepdims=True)
    wt = jnp.where((eidx == i1) | (eidx == i2), gates, 0.0)
    return jnp.transpose(wt)


def _mix(feats, w, ew_ref, eb_ref):
    out = jnp.dot(w, eb_ref[:], preferred_element_type=jnp.float32)
    for e in range(E):
        out = out + jnp.dot(w[:, e:e + 1] * feats[e], ew_ref[e],
                            preferred_element_type=jnp.float32)
    return out


def _moe_body(feat_ref, gw_ref, gb_ref, ew_ref, eb_ref, out_ref, *, bn):
    nsub = 4
    h = bn // nsub
    fl = [None] * nsub
    lg = [None] * nsub
    fl[0], lg[0] = _gating(feat_ref, gw_ref, 0, h)
    fl[1], lg[1] = _gating(feat_ref, gw_ref, h, h)
    w = _topk_mask(lg[0], gb_ref, h)
    for j in range(nsub):
        if j + 2 < nsub:
            fl[j + 2], lg[j + 2] = _gating(feat_ref, gw_ref, (j + 2) * h, h)
        out = _mix(fl[j], w, ew_ref, eb_ref)
        if j + 1 < nsub:
            w = _topk_mask(lg[j + 1], gb_ref, h)
        out_ref[j * h:(j + 1) * h, :] = out


@jax.jit
def kernel(feat_list, gate_W, gate_b, expert_W, expert_b):
    E_, N_, D_ = feat_list.shape
    F = expert_W.shape[2]
    bn = 2048
    grid = (N_ // bn,)

    body = functools.partial(_moe_body, bn=bn)
    return pl.pallas_call(
        body,
        grid=grid,
        in_specs=[
            pl.BlockSpec((E_, bn, D_), lambda i: (0, i, 0)),
            pl.BlockSpec((E_ * D_, E_), lambda i: (0, 0)),
            pl.BlockSpec((E_, 1), lambda i: (0, 0)),
            pl.BlockSpec((E_, D_, F), lambda i: (0, 0, 0)),
            pl.BlockSpec((E_, F), lambda i: (0, 0)),
        ],
        out_specs=pl.BlockSpec((bn, F), lambda i: (i, 0)),
        out_shape=jax.ShapeDtypeStruct((N_, F), jnp.float32),
        compiler_params=pltpu.CompilerParams(
            dimension_semantics=("arbitrary",),
        ),
    )(feat_list, gate_W, gate_b.reshape(E_, 1), expert_W, expert_b)


# speedup vs baseline: 1.3011x; 1.0107x over previous
import functools

import jax
import jax.numpy as jnp
from jax.experimental import pallas as pl
from jax.experimental.pallas import tpu as pltpu

E = 8
D = 128
FUSION = 128


def _gating(feat_ref, gw_ref, lo, bn):
    feats = [feat_ref[e, lo:lo + bn] for e in range(E)]
    xu = jnp.concatenate(feats, axis=1)
    logits = jnp.dot(xu, gw_ref[:], preferred_element_type=jnp.float32)
    return feats, logits


def _topk_mask(logits, gb_ref, bn):
    lt = jnp.transpose(logits) + gb_ref[:]
    m = jnp.max(lt, axis=0, keepdims=True)
    p = jnp.exp(lt - m)
    gates = p * (1.0 / jnp.sum(p, axis=0, keepdims=True))

    eidx = jax.lax.broadcasted_iota(jnp.int32, (E, bn), 0)
    m1 = jnp.max(gates, axis=0, keepdims=True)
    i1 = jnp.min(jnp.where(gates == m1, eidx, E), axis=0, keepdims=True)
    g2 = jnp.where(eidx == i1, -jnp.inf, gates)
    m2 = jnp.max(g2, axis=0, keepdims=True)
    i2 = jnp.min(jnp.where(g2 == m2, eidx, E), axis=0, keepdims=True)
    wt = jnp.where((eidx == i1) | (eidx == i2), gates, 0.0)
    return jnp.transpose(wt)


def _mix(feats, w, ew_ref, eb_ref):
    out = jnp.dot(w, eb_ref[:], preferred_element_type=jnp.float32)
    for e in range(E):
        out = out + jnp.dot(w[:, e:e + 1] * feats[e], ew_ref[e],
                            preferred_element_type=jnp.float32)
    return out


def _moe_body(feat_ref, gw_ref, gb_ref, ew_ref, eb_ref, out_ref, *, bn):
    nsub = 8
    h = bn // nsub
    fl = [None] * nsub
    lg = [None] * nsub
    fl[0], lg[0] = _gating(feat_ref, gw_ref, 0, h)
    fl[1], lg[1] = _gating(feat_ref, gw_ref, h, h)
    w = _topk_mask(lg[0], gb_ref, h)
    for j in range(nsub):
        if j + 2 < nsub:
            fl[j + 2], lg[j + 2] = _gating(feat_ref, gw_ref, (j + 2) * h, h)
        out = _mix(fl[j], w, ew_ref, eb_ref)
        if j + 1 < nsub:
            w = _topk_mask(lg[j + 1], gb_ref, h)
        out_ref[j * h:(j + 1) * h, :] = out


@jax.jit
def kernel(feat_list, gate_W, gate_b, expert_W, expert_b):
    E_, N_, D_ = feat_list.shape
    F = expert_W.shape[2]
    bn = 4096
    grid = (N_ // bn,)

    body = functools.partial(_moe_body, bn=bn)
    return pl.pallas_call(
        body,
        grid=grid,
        in_specs=[
            pl.BlockSpec((E_, bn, D_), lambda i: (0, i, 0)),
            pl.BlockSpec((E_ * D_, E_), lambda i: (0, 0)),
            pl.BlockSpec((E_, 1), lambda i: (0, 0)),
            pl.BlockSpec((E_, D_, F), lambda i: (0, 0, 0)),
            pl.BlockSpec((E_, F), lambda i: (0, 0)),
        ],
        out_specs=pl.BlockSpec((bn, F), lambda i: (i, 0)),
        out_shape=jax.ShapeDtypeStruct((N_, F), jnp.float32),
        compiler_params=pltpu.CompilerParams(
            dimension_semantics=("arbitrary",),
        ),
    )(feat_list, gate_W, gate_b.reshape(E_, 1), expert_W, expert_b)
